# baseline (speedup 1.0000x reference)
import functools

import jax
import jax.numpy as jnp
from jax import lax
from jax.experimental import pallas as pl
from jax.experimental.pallas import tpu as pltpu
from jax.experimental.pallas import tpu_sc as plsc

N = 10000
NP = 10240
D = 128
DH = D // 2
K = 10
NTILE = 16
RPT = NP // NTILE
CHUNK = 128
SUB = 2
PKBITS = 14
PKMASK = (1 << PKBITS) - 1


def _bn_body(x_ref, g_ref, b_ref, lo_ref, hi_ref):
    x = x_ref[...]
    mu = jnp.mean(x, axis=0, keepdims=True)
    xc = x - mu
    var = jnp.mean(xc * xc, axis=0, keepdims=True)
    xb = xc * (g_ref[...] / jnp.sqrt(var + 1e-5)) + b_ref[...]
    zpad = jnp.zeros((NP - N, D), jnp.float32)
    xbp = jnp.concatenate([xb, zpad], axis=0)
    lo_ref[...] = xbp[:, :DH]
    hi_ref[...] = xbp[:, DH:]


def _batchnorm(x, gamma, beta):
    return pl.pallas_call(
        _bn_body,
        out_shape=(
            jax.ShapeDtypeStruct((NP, DH), jnp.float32),
            jax.ShapeDtypeStruct((NP, DH), jnp.float32),
        ),
    )(x, gamma.reshape(1, D), beta.reshape(1, D))


def _mlp_body(lo_ref, hi_ref, w1a_ref, w1b_ref, b1_ref, a1_ref,
              w2_ref, b2_ref, a2_ref, w3_ref, b3_ref, out_ref):
    f32 = jnp.float32
    h1 = (lax.dot_general(lo_ref[...], w1a_ref[...], (((1,), (0,)), ((), ())),
                          preferred_element_type=f32)
          + lax.dot_general(hi_ref[...], w1b_ref[...], (((1,), (0,)), ((), ())),
                            preferred_element_type=f32)
          + b1_ref[...])
    a1 = a1_ref[0, 0]
    h1 = jnp.where(h1 >= 0, h1, a1 * h1)
    h2 = lax.dot_general(h1, w2_ref[...], (((1,), (0,)), ((), ())),
                         preferred_element_type=f32) + b2_ref[...]
    a2 = a2_ref[0, 0]
    h2 = jnp.where(h2 >= 0, h2, a2 * h2)
    logits = lax.dot_general(h2, w3_ref[...], (((1,), (0,)), ((), ())),
                             preferred_element_type=f32) + b3_ref[...]
    lane = lax.broadcasted_iota(jnp.int32, logits.shape, 1)
    neg = jnp.float32(-1e30)
    masked = jnp.where(lane < 2, logits, neg)
    m = jnp.max(masked, axis=1, keepdims=True)
    e = jnp.where(lane < 2, jnp.exp(masked - m), 0.0)
    lse = m + jnp.log(jnp.sum(e, axis=1, keepdims=True))
    res = logits - lse
    out_ref[...] = res[:, :2]


def _mlp(lo, hi, w1, b1, a1, w2, b2, a2, w3, b3):
    w3p = jnp.zeros((64, 128), jnp.float32).at[:, :2].set(w3)
    b3p = jnp.zeros((1, 128), jnp.float32).at[0, :2].set(b3)
    return pl.pallas_call(
        _mlp_body,
        out_shape=jax.ShapeDtypeStruct((N, 2), jnp.float32),
    )(lo, hi, w1[:DH], w1[DH:], b1.reshape(1, 64), a1.reshape(1, 1),
      w2, b2.reshape(1, 64), a2.reshape(1, 1), w3p, b3p)


def _scale2(g, off, sb, w_v, ch):
    @pl.loop(0, CHUNK // 16)
    def _grp(g16):
        wv16 = w_v[ch, pl.ds(g16 * 16, 16)]
        for e in range(16):
            w1 = jnp.full((16,), wv16[e], jnp.float32)
            j = g16 * 16 + e
            for q in range(DH // 16):
                sl = pl.ds(q * 16, 16)
                sb[j, sl] = g[off + j, sl] * w1


def _unpack_cols(pk_v, c, gidx, slot):
    for q in range(CHUNK // 16):
        sl = pl.ds(q * 16, 16)
        gidx[slot, sl] = jnp.bitwise_and(pk_v[c, sl], PKMASK)


def _unpack_rows(pk_v, c, sidx, slot):
    for q in range(CHUNK // 16):
        sl = pl.ds(q * 16, 16)
        sidx[slot, sl] = jax.lax.shift_right_logical(pk_v[c, sl], PKBITS)


def _prop_tile(x_in, work, zeros_hbm, pk_v, w_v, gidx, sidx,
               gb, sbuf, y_sp, gsem, ssem, s):
    nch = w_v.shape[0]
    rs = pl.ds(s * RPT, RPT)

    for k in range(RPT // CHUNK):
        part = pl.ds(s * RPT + k * CHUNK, CHUNK)
        stage = gb[0].at[pl.ds(0, CHUNK)]
        pltpu.sync_copy(x_in.at[part], stage)
        pltpu.sync_copy(stage, work.at[part])
    plsc.subcore_barrier()

    @pl.loop(0, K)
    def _iter(_):
        pltpu.sync_copy(zeros_hbm.at[rs], y_sp.at[rs])
        plsc.subcore_barrier()

        for b in range(2):
            for u in range(SUB):
                gslot = b * SUB + u
                _unpack_cols(pk_v, gslot, gidx, gslot)
                pltpu.async_copy(work.at[gidx.at[gslot]],
                                 gb[b].at[pl.ds(u * CHUNK, CHUNK)], gsem[b])

        @pl.loop(0, nch, step=2 * SUB)
        def _slots(ch):
            for b in range(2):
                base = ch + b * SUB
                g = gb[b]
                for u in range(SUB):
                    pltpu.make_async_copy(
                        work.at[gidx.at[b * SUB + u]],
                        g.at[pl.ds(u * CHUNK, CHUNK)], gsem[b]).wait()

                for u in range(SUB):
                    c = base + u
                    sb = sbuf[u]

                    @pl.when(c >= 2)
                    def _():
                        pltpu.make_async_copy(
                            sb, y_sp.at[sidx.at[u]], ssem[u]).wait()

                    _scale2(g, u * CHUNK, sb, w_v, c)
                    _unpack_rows(pk_v, c, sidx, u)
                    pltpu.async_copy(sb, y_sp.at[sidx.at[u]], ssem[u],
                                     add=True)

                for u in range(SUB):
                    gslot = b * SUB + u
                    _unpack_cols(pk_v, base + 2 * SUB + u, gidx, gslot)
                    pltpu.async_copy(work.at[gidx.at[gslot]],
                                     g.at[pl.ds(u * CHUNK, CHUNK)], gsem[b])

        for b in range(2):
            for u in range(SUB):
                pltpu.make_async_copy(
                    work.at[gidx.at[b * SUB + u]],
                    gb[b].at[pl.ds(u * CHUNK, CHUNK)], gsem[b]).wait()
        for u in range(SUB):
            pltpu.make_async_copy(
                sbuf[u], y_sp.at[sidx.at[u]], ssem[u]).wait()

        plsc.subcore_barrier()
        pltpu.sync_copy(y_sp.at[rs], work.at[rs])
        plsc.subcore_barrier()


def _propagate(xb_lo, xb_hi, pk3, w3, zeros):
    nch = w3.shape[1]
    mesh = plsc.VectorSubcoreMesh(core_axis_name="c", subcore_axis_name="s")

    @functools.partial(
        pl.kernel,
        out_type=(
            jax.ShapeDtypeStruct((NP, DH), jnp.float32),
            jax.ShapeDtypeStruct((NP, DH), jnp.float32),
        ),
        mesh=mesh,
        compiler_params=pltpu.CompilerParams(use_tc_tiling_on_sc=False),
        scratch_types=[
            pltpu.VMEM((nch + 2 * SUB, CHUNK), jnp.int32),
            pltpu.VMEM((nch, CHUNK), jnp.float32),
            pltpu.VMEM((2 * SUB, CHUNK), jnp.int32),
            pltpu.VMEM((SUB, CHUNK), jnp.int32),
            [pltpu.VMEM((SUB * CHUNK, DH), jnp.float32)] * 2,
            [pltpu.VMEM((CHUNK, DH), jnp.float32)] * SUB,
            pltpu.VMEM_SHARED((NP, DH), jnp.float32),
            [pltpu.SemaphoreType.DMA] * 2,
            [pltpu.SemaphoreType.DMA] * SUB,
        ],
    )
    def prop(xlo, xhi, pk_hbm, w_hbm, zeros_hbm,
             out_lo, out_hi, pk_v, w_v, gidx, sidx, gb, sbuf, y_sp,
             gsem, ssem):
        c = lax.axis_index("c")
        s = lax.axis_index("s")
        pltpu.sync_copy(pk_hbm.at[s], pk_v)
        pltpu.sync_copy(w_hbm.at[s], w_v)

        @pl.when(c == 0)
        def _():
            _prop_tile(xlo, out_lo, zeros_hbm, pk_v, w_v, gidx, sidx,
                       gb, sbuf, y_sp, gsem, ssem, s)

        @pl.when(c == 1)
        def _():
            _prop_tile(xhi, out_hi, zeros_hbm, pk_v, w_v, gidx, sidx,
                       gb, sbuf, y_sp, gsem, ssem, s)

    return prop(xb_lo, xb_hi, pk3, w3, zeros)


def kernel(x, x_cov, edge_index, edge_weight, norm_weight, bn_gamma, bn_beta,
           p0_W, p0_C, p1_W, p1_C, feature_corr,
           mlp_W1, mlp_b1, mlp_a1, mlp_W2, mlp_b2, mlp_a2, mlp_W3, mlp_b3):
    E = edge_index.shape[1]
    quant = 2 * SUB * CHUNK
    ept = ((E + NTILE * quant - 1) // (NTILE * quant)) * quant
    epad = NTILE * ept
    pad = epad - E
    rows = jnp.concatenate([edge_index[0], jnp.zeros((pad,), edge_index.dtype)])
    cols = jnp.concatenate([edge_index[1], jnp.zeros((pad,), edge_index.dtype)])
    w = jnp.concatenate([norm_weight, jnp.zeros((pad,), norm_weight.dtype)])
    nch = ept // CHUNK
    packed = (rows.astype(jnp.int32) << PKBITS) | cols.astype(jnp.int32)
    pk3 = packed.reshape(NTILE, nch, CHUNK)
    pk3 = jnp.pad(pk3, ((0, 0), (0, 2 * SUB), (0, 0)))
    w3 = w.reshape(NTILE, nch, CHUNK)
    zeros = jnp.zeros((NP, DH), jnp.float32)

    xb_lo, xb_hi = _batchnorm(x, bn_gamma, bn_beta)
    xh_lo, xh_hi = _propagate(xb_lo, xb_hi, pk3, w3, zeros)
    return _mlp(xh_lo[:N], xh_hi[:N], mlp_W1, mlp_b1, mlp_a1,
                mlp_W2, mlp_b2, mlp_a2, mlp_W3, mlp_b3)


# speedup vs baseline: 1.0010x; 1.0010x over previous
import functools

import jax
import jax.numpy as jnp
from jax import lax
from jax.experimental import pallas as pl
from jax.experimental.pallas import tpu as pltpu
from jax.experimental.pallas import tpu_sc as plsc

N = 10000
NP = 10240
D = 128
DH = D // 2
K = 10
NTILE = 16
RPT = NP // NTILE
CHUNK = 64
GSUB = 4
SRING = 8
PKBITS = 14
PKMASK = (1 << PKBITS) - 1


def _bn_body(x_ref, g_ref, b_ref, lo_ref, hi_ref):
    x = x_ref[...]
    mu = jnp.mean(x, axis=0, keepdims=True)
    xc = x - mu
    var = jnp.mean(xc * xc, axis=0, keepdims=True)
    xb = xc * (g_ref[...] / jnp.sqrt(var + 1e-5)) + b_ref[...]
    zpad = jnp.zeros((NP - N, D), jnp.float32)
    xbp = jnp.concatenate([xb, zpad], axis=0)
    lo_ref[...] = xbp[:, :DH]
    hi_ref[...] = xbp[:, DH:]


def _batchnorm(x, gamma, beta):
    return pl.pallas_call(
        _bn_body,
        out_shape=(
            jax.ShapeDtypeStruct((NP, DH), jnp.float32),
            jax.ShapeDtypeStruct((NP, DH), jnp.float32),
        ),
    )(x, gamma.reshape(1, D), beta.reshape(1, D))


def _mlp_body(lo_ref, hi_ref, w1a_ref, w1b_ref, b1_ref, a1_ref,
              w2_ref, b2_ref, a2_ref, w3_ref, b3_ref, out_ref):
    f32 = jnp.float32
    h1 = (lax.dot_general(lo_ref[...], w1a_ref[...], (((1,), (0,)), ((), ())),
                          preferred_element_type=f32)
          + lax.dot_general(hi_ref[...], w1b_ref[...], (((1,), (0,)), ((), ())),
                            preferred_element_type=f32)
          + b1_ref[...])
    a1 = a1_ref[0, 0]
    h1 = jnp.where(h1 >= 0, h1, a1 * h1)
    h2 = lax.dot_general(h1, w2_ref[...], (((1,), (0,)), ((), ())),
                         preferred_element_type=f32) + b2_ref[...]
    a2 = a2_ref[0, 0]
    h2 = jnp.where(h2 >= 0, h2, a2 * h2)
    logits = lax.dot_general(h2, w3_ref[...], (((1,), (0,)), ((), ())),
                             preferred_element_type=f32) + b3_ref[...]
    lane = lax.broadcasted_iota(jnp.int32, logits.shape, 1)
    neg = jnp.float32(-1e30)
    masked = jnp.where(lane < 2, logits, neg)
    m = jnp.max(masked, axis=1, keepdims=True)
    e = jnp.where(lane < 2, jnp.exp(masked - m), 0.0)
    lse = m + jnp.log(jnp.sum(e, axis=1, keepdims=True))
    res = logits - lse
    out_ref[...] = res[:, :2]


def _mlp(lo, hi, w1, b1, a1, w2, b2, a2, w3, b3):
    w3p = jnp.zeros((64, 128), jnp.float32).at[:, :2].set(w3)
    b3p = jnp.zeros((1, 128), jnp.float32).at[0, :2].set(b3)
    return pl.pallas_call(
        _mlp_body,
        out_shape=jax.ShapeDtypeStruct((N, 2), jnp.float32),
    )(lo, hi, w1[:DH], w1[DH:], b1.reshape(1, 64), a1.reshape(1, 1),
      w2, b2.reshape(1, 64), a2.reshape(1, 1), w3p, b3p)


def _scale2(g, off, sb, w_v, ch):
    @pl.loop(0, CHUNK // 16)
    def _grp(g16):
        wv16 = w_v[ch, pl.ds(g16 * 16, 16)]
        for e in range(16):
            w1 = jnp.full((16,), wv16[e], jnp.float32)
            j = g16 * 16 + e
            for q in range(DH // 16):
                sl = pl.ds(q * 16, 16)
                sb[j, sl] = g[off + j, sl] * w1


def _unpack_cols(pk_v, c, gidx, slot):
    for q in range(CHUNK // 16):
        sl = pl.ds(q * 16, 16)
        gidx[slot, sl] = jnp.bitwise_and(pk_v[c, sl], PKMASK)


def _unpack_rows(pk_v, c, sidx, slot):
    for q in range(CHUNK // 16):
        sl = pl.ds(q * 16, 16)
        sidx[slot, sl] = jax.lax.shift_right_logical(pk_v[c, sl], PKBITS)


def _prop_tile(x_in, work, zeros_hbm, pk_v, w_v, gidx, sidx,
               gb, sbuf, y_sp, gsem, ssem, s):
    nch = w_v.shape[0]
    rs = pl.ds(s * RPT, RPT)

    for k in range(RPT // CHUNK):
        part = pl.ds(s * RPT + k * CHUNK, CHUNK)
        stage = gb[0].at[pl.ds(0, CHUNK)]
        pltpu.sync_copy(x_in.at[part], stage)
        pltpu.sync_copy(stage, work.at[part])
    plsc.subcore_barrier()

    @pl.loop(0, K)
    def _iter(_):
        pltpu.sync_copy(zeros_hbm.at[rs], y_sp.at[rs])
        plsc.subcore_barrier()

        for b in range(2):
            for u in range(GSUB):
                gslot = b * GSUB + u
                _unpack_cols(pk_v, gslot, gidx, gslot)
                pltpu.async_copy(work.at[gidx.at[gslot]],
                                 gb[b].at[pl.ds(u * CHUNK, CHUNK)], gsem[b])

        @pl.loop(0, nch, step=2 * GSUB)
        def _slots(ch):
            for b in range(2):
                base = ch + b * GSUB
                g = gb[b]
                for u in range(GSUB):
                    pltpu.make_async_copy(
                        work.at[gidx.at[b * GSUB + u]],
                        g.at[pl.ds(u * CHUNK, CHUNK)], gsem[b]).wait()

                for u in range(GSUB):
                    c = base + u
                    p = b * GSUB + u
                    sb = sbuf[p]

                    @pl.when(c >= SRING)
                    def _():
                        pltpu.make_async_copy(
                            sb, y_sp.at[sidx.at[p]], ssem[p]).wait()

                    _scale2(g, u * CHUNK, sb, w_v, c)
                    _unpack_rows(pk_v, c, sidx, p)
                    pltpu.async_copy(sb, y_sp.at[sidx.at[p]], ssem[p],
                                     add=True)

                for u in range(GSUB):
                    gslot = b * GSUB + u
                    _unpack_cols(pk_v, base + 2 * GSUB + u, gidx, gslot)
                    pltpu.async_copy(work.at[gidx.at[gslot]],
                                     g.at[pl.ds(u * CHUNK, CHUNK)], gsem[b])

        for b in range(2):
            for u in range(GSUB):
                pltpu.make_async_copy(
                    work.at[gidx.at[b * GSUB + u]],
                    gb[b].at[pl.ds(u * CHUNK, CHUNK)], gsem[b]).wait()
        for p in range(SRING):
            pltpu.make_async_copy(
                sbuf[p], y_sp.at[sidx.at[p]], ssem[p]).wait()

        plsc.subcore_barrier()
        pltpu.sync_copy(y_sp.at[rs], work.at[rs])
        plsc.subcore_barrier()


def _propagate(xb_lo, xb_hi, pk3, w3, zeros):
    nch = w3.shape[1]
    mesh = plsc.VectorSubcoreMesh(core_axis_name="c", subcore_axis_name="s")

    @functools.partial(
        pl.kernel,
        out_type=(
            jax.ShapeDtypeStruct((NP, DH), jnp.float32),
            jax.ShapeDtypeStruct((NP, DH), jnp.float32),
        ),
        mesh=mesh,
        compiler_params=pltpu.CompilerParams(use_tc_tiling_on_sc=False),
        scratch_types=[
            pltpu.VMEM((nch + 2 * GSUB, CHUNK), jnp.int32),
            pltpu.VMEM((nch, CHUNK), jnp.float32),
            pltpu.VMEM((2 * GSUB, CHUNK), jnp.int32),
            pltpu.VMEM((SRING, CHUNK), jnp.int32),
            [pltpu.VMEM((GSUB * CHUNK, DH), jnp.float32)] * 2,
            [pltpu.VMEM((CHUNK, DH), jnp.float32)] * SRING,
            pltpu.VMEM_SHARED((NP, DH), jnp.float32),
            [pltpu.SemaphoreType.DMA] * 2,
            [pltpu.SemaphoreType.DMA] * SRING,
        ],
    )
    def prop(xlo, xhi, pk_hbm, w_hbm, zeros_hbm,
             out_lo, out_hi, pk_v, w_v, gidx, sidx, gb, sbuf, y_sp,
             gsem, ssem):
        c = lax.axis_index("c")
        s = lax.axis_index("s")
        pltpu.sync_copy(pk_hbm.at[s], pk_v)
        pltpu.sync_copy(w_hbm.at[s], w_v)

        @pl.when(c == 0)
        def _():
            _prop_tile(xlo, out_lo, zeros_hbm, pk_v, w_v, gidx, sidx,
                       gb, sbuf, y_sp, gsem, ssem, s)

        @pl.when(c == 1)
        def _():
            _prop_tile(xhi, out_hi, zeros_hbm, pk_v, w_v, gidx, sidx,
                       gb, sbuf, y_sp, gsem, ssem, s)

    return prop(xb_lo, xb_hi, pk3, w3, zeros)


def kernel(x, x_cov, edge_index, edge_weight, norm_weight, bn_gamma, bn_beta,
           p0_W, p0_C, p1_W, p1_C, feature_corr,
           mlp_W1, mlp_b1, mlp_a1, mlp_W2, mlp_b2, mlp_a2, mlp_W3, mlp_b3):
    E = edge_index.shape[1]
    quant = 2 * GSUB * CHUNK
    ept = ((E + NTILE * quant - 1) // (NTILE * quant)) * quant
    epad = NTILE * ept
    pad = epad - E
    rows = jnp.concatenate([edge_index[0], jnp.zeros((pad,), edge_index.dtype)])
    cols = jnp.concatenate([edge_index[1], jnp.zeros((pad,), edge_index.dtype)])
    w = jnp.concatenate([norm_weight, jnp.zeros((pad,), norm_weight.dtype)])
    nch = ept // CHUNK
    packed = (rows.astype(jnp.int32) << PKBITS) | cols.astype(jnp.int32)
    pk3 = packed.reshape(NTILE, nch, CHUNK)
    pk3 = jnp.pad(pk3, ((0, 0), (0, 2 * GSUB), (0, 0)))
    w3 = w.reshape(NTILE, nch, CHUNK)
    zeros = jnp.zeros((NP, DH), jnp.float32)

    xb_lo, xb_hi = _batchnorm(x, bn_gamma, bn_beta)
    xh_lo, xh_hi = _propagate(xb_lo, xb_hi, pk3, w3, zeros)
    return _mlp(xh_lo[:N], xh_hi[:N], mlp_W1, mlp_b1, mlp_a1,
                mlp_W2, mlp_b2, mlp_a2, mlp_W3, mlp_b3)


# speedup vs baseline: 4.4799x; 4.4753x over previous
import functools

import jax
import jax.numpy as jnp
from jax import lax
from jax.experimental import pallas as pl
from jax.experimental.pallas import tpu as pltpu
from jax.experimental.pallas import tpu_sc as plsc

N = 10000
NP = 10240
D = 128
DH = D // 2
K = 10
NTILE = 16
RPT = NP // NTILE
CHUNK = 64
GSUB = 4
SRING = 8
PKBITS = 14
PKMASK = (1 << PKBITS) - 1


def _bn_body(x_ref, g_ref, b_ref, lo_ref, hi_ref):
    x = x_ref[...]
    mu = jnp.mean(x, axis=0, keepdims=True)
    xc = x - mu
    var = jnp.mean(xc * xc, axis=0, keepdims=True)
    xb = xc * (g_ref[...] / jnp.sqrt(var + 1e-5)) + b_ref[...]
    zpad = jnp.zeros((NP - N, D), jnp.float32)
    xbp = jnp.concatenate([xb, zpad], axis=0)
    lo_ref[...] = xbp[:, :DH]
    hi_ref[...] = xbp[:, DH:]


def _batchnorm(x, gamma, beta):
    return pl.pallas_call(
        _bn_body,
        out_shape=(
            jax.ShapeDtypeStruct((NP, DH), jnp.float32),
            jax.ShapeDtypeStruct((NP, DH), jnp.float32),
        ),
    )(x, gamma.reshape(1, D), beta.reshape(1, D))


def _mlp_body(lo_ref, hi_ref, w1a_ref, w1b_ref, b1_ref, a1_ref,
              w2_ref, b2_ref, a2_ref, w3_ref, b3_ref, out_ref):
    f32 = jnp.float32
    h1 = (lax.dot_general(lo_ref[...], w1a_ref[...], (((1,), (0,)), ((), ())),
                          preferred_element_type=f32)
          + lax.dot_general(hi_ref[...], w1b_ref[...], (((1,), (0,)), ((), ())),
                            preferred_element_type=f32)
          + b1_ref[...])
    a1 = a1_ref[0, 0]
    h1 = jnp.where(h1 >= 0, h1, a1 * h1)
    h2 = lax.dot_general(h1, w2_ref[...], (((1,), (0,)), ((), ())),
                         preferred_element_type=f32) + b2_ref[...]
    a2 = a2_ref[0, 0]
    h2 = jnp.where(h2 >= 0, h2, a2 * h2)
    logits = lax.dot_general(h2, w3_ref[...], (((1,), (0,)), ((), ())),
                             preferred_element_type=f32) + b3_ref[...]
    lane = lax.broadcasted_iota(jnp.int32, logits.shape, 1)
    neg = jnp.float32(-1e30)
    masked = jnp.where(lane < 2, logits, neg)
    m = jnp.max(masked, axis=1, keepdims=True)
    e = jnp.where(lane < 2, jnp.exp(masked - m), 0.0)
    lse = m + jnp.log(jnp.sum(e, axis=1, keepdims=True))
    res = logits - lse
    out_ref[...] = res[:, :2]


def _mlp(lo, hi, w1, b1, a1, w2, b2, a2, w3, b3):
    w3p = jnp.zeros((64, 128), jnp.float32).at[:, :2].set(w3)
    b3p = jnp.zeros((1, 128), jnp.float32).at[0, :2].set(b3)
    return pl.pallas_call(
        _mlp_body,
        out_shape=jax.ShapeDtypeStruct((N, 2), jnp.float32),
    )(lo, hi, w1[:DH], w1[DH:], b1.reshape(1, 64), a1.reshape(1, 1),
      w2, b2.reshape(1, 64), a2.reshape(1, 1), w3p, b3p)


def _scale2(g, off, sb, w_v, ch):
    @pl.loop(0, CHUNK // 16)
    def _grp(g16):
        wv16 = w_v[ch, pl.ds(g16 * 16, 16)]
        for e in range(16):
            w1 = jnp.full((16,), wv16[e], jnp.float32)
            j = g16 * 16 + e
            for q in range(DH // 16):
                sl = pl.ds(q * 16, 16)
                sb[j, sl] = g[off + j, sl] * w1


def _unpack_cols(pk_v, c, gidx, slot):
    for q in range(CHUNK // 16):
        sl = pl.ds(q * 16, 16)
        gidx[slot, sl] = jnp.bitwise_and(pk_v[c, sl], PKMASK)


def _unpack_rows(pk_v, c, sidx, slot):
    for q in range(CHUNK // 16):
        sl = pl.ds(q * 16, 16)
        sidx[slot, sl] = jax.lax.shift_right_logical(pk_v[c, sl], PKBITS)


def _prop_tile(x_in, work, zeros_hbm, pk_v, w_v, gidx, sidx,
               gb, sbuf, y_sp, gsem, ssem, s):
    nch = w_v.shape[0]
    rs = pl.ds(s * RPT, RPT)

    for k in range(RPT // CHUNK):
        part = pl.ds(s * RPT + k * CHUNK, CHUNK)
        stage = gb[0].at[pl.ds(0, CHUNK)]
        pltpu.sync_copy(x_in.at[part], stage)
        pltpu.sync_copy(stage, work.at[part])
    plsc.subcore_barrier()

    @pl.loop(0, K)
    def _iter(_):
        pltpu.sync_copy(zeros_hbm.at[rs], y_sp.at[rs])
        plsc.subcore_barrier()

        for b in range(2):
            for u in range(GSUB):
                gslot = b * GSUB + u
                _unpack_cols(pk_v, gslot, gidx, gslot)
                pltpu.async_copy(work.at[gidx.at[gslot]],
                                 gb[b].at[pl.ds(u * CHUNK, CHUNK)], gsem[b])

        @pl.loop(0, nch, step=2 * GSUB)
        def _slots(ch):
            for b in range(2):
                base = ch + b * GSUB
                g = gb[b]
                for u in range(GSUB):
                    pltpu.make_async_copy(
                        work.at[gidx.at[b * GSUB + u]],
                        g.at[pl.ds(u * CHUNK, CHUNK)], gsem[b]).wait()

                for u in range(GSUB):
                    c = base + u
                    p = b * GSUB + u
                    sb = sbuf[p]

                    @pl.when(c >= SRING)
                    def _():
                        pltpu.make_async_copy(
                            sb, y_sp.at[sidx.at[p]], ssem[p]).wait()

                    _scale2(g, u * CHUNK, sb, w_v, c)
                    _unpack_rows(pk_v, c, sidx, p)
                    pltpu.async_copy(sb, y_sp.at[sidx.at[p]], ssem[p],
                                     add=True)

                for u in range(GSUB):
                    gslot = b * GSUB + u
                    _unpack_cols(pk_v, base + 2 * GSUB + u, gidx, gslot)
                    pltpu.async_copy(work.at[gidx.at[gslot]],
                                     g.at[pl.ds(u * CHUNK, CHUNK)], gsem[b])

        for b in range(2):
            for u in range(GSUB):
                pltpu.make_async_copy(
                    work.at[gidx.at[b * GSUB + u]],
                    gb[b].at[pl.ds(u * CHUNK, CHUNK)], gsem[b]).wait()
        for p in range(SRING):
            pltpu.make_async_copy(
                sbuf[p], y_sp.at[sidx.at[p]], ssem[p]).wait()

        plsc.subcore_barrier()
        pltpu.sync_copy(y_sp.at[rs], work.at[rs])
        plsc.subcore_barrier()


def _propagate(xb_lo, xb_hi, pk3, w3, zeros):
    nch = w3.shape[1]
    mesh = plsc.VectorSubcoreMesh(core_axis_name="c", subcore_axis_name="s")

    @functools.partial(
        pl.kernel,
        out_type=(
            jax.ShapeDtypeStruct((NP, DH), jnp.float32),
            jax.ShapeDtypeStruct((NP, DH), jnp.float32),
        ),
        mesh=mesh,
        compiler_params=pltpu.CompilerParams(use_tc_tiling_on_sc=False),
        scratch_types=[
            pltpu.VMEM((nch + 2 * GSUB, CHUNK), jnp.int32),
            pltpu.VMEM((nch, CHUNK), jnp.float32),
            pltpu.VMEM((2 * GSUB, CHUNK), jnp.int32),
            pltpu.VMEM((SRING, CHUNK), jnp.int32),
            [pltpu.VMEM((GSUB * CHUNK, DH), jnp.float32)] * 2,
            [pltpu.VMEM((CHUNK, DH), jnp.float32)] * SRING,
            pltpu.VMEM_SHARED((NP, DH), jnp.float32),
            [pltpu.SemaphoreType.DMA] * 2,
            [pltpu.SemaphoreType.DMA] * SRING,
        ],
    )
    def prop(xlo, xhi, pk_hbm, w_hbm, zeros_hbm,
             out_lo, out_hi, pk_v, w_v, gidx, sidx, gb, sbuf, y_sp,
             gsem, ssem):
        c = lax.axis_index("c")
        s = lax.axis_index("s")
        pltpu.sync_copy(pk_hbm.at[s], pk_v)
        pltpu.sync_copy(w_hbm.at[s], w_v)

        @pl.when(c == 0)
        def _():
            _prop_tile(xlo, out_lo, zeros_hbm, pk_v, w_v, gidx, sidx,
                       gb, sbuf, y_sp, gsem, ssem, s)

        @pl.when(c == 1)
        def _():
            _prop_tile(xhi, out_hi, zeros_hbm, pk_v, w_v, gidx, sidx,
                       gb, sbuf, y_sp, gsem, ssem, s)

    return prop(xb_lo, xb_hi, pk3, w3, zeros)


def kernel(x, x_cov, edge_index, edge_weight, norm_weight, bn_gamma, bn_beta,
           p0_W, p0_C, p1_W, p1_C, feature_corr,
           mlp_W1, mlp_b1, mlp_a1, mlp_W2, mlp_b2, mlp_a2, mlp_W3, mlp_b3):
    E = edge_index.shape[1]
    quant = 2 * GSUB * CHUNK
    ept = ((E + NTILE * quant - 1) // (NTILE * quant)) * quant
    epad = NTILE * ept
    pad = epad - E
    spread = (jnp.arange(pad, dtype=jnp.int32) * 37) % N
    rows = jnp.concatenate([edge_index[0].astype(jnp.int32), spread])
    cols = jnp.concatenate([edge_index[1].astype(jnp.int32), spread])
    w = jnp.concatenate([norm_weight, jnp.zeros((pad,), norm_weight.dtype)])
    nch = ept // CHUNK
    packed = (rows << PKBITS) | cols
    pk3 = packed.reshape(NTILE, nch, CHUNK)
    gpad = (jnp.arange(NTILE * 2 * GSUB * CHUNK, dtype=jnp.int32) * 53) % N
    gpad = gpad.reshape(NTILE, 2 * GSUB, CHUNK)
    pk3 = jnp.concatenate([pk3, (gpad << PKBITS) | gpad], axis=1)
    w3 = w.reshape(NTILE, nch, CHUNK)
    zeros = jnp.zeros((NP, DH), jnp.float32)

    xb_lo, xb_hi = _batchnorm(x, bn_gamma, bn_beta)
    xh_lo, xh_hi = _propagate(xb_lo, xb_hi, pk3, w3, zeros)
    return _mlp(xh_lo[:N], xh_hi[:N], mlp_W1, mlp_b1, mlp_a1,
                mlp_W2, mlp_b2, mlp_a2, mlp_W3, mlp_b3)


# speedup vs baseline: 5.0249x; 1.1217x over previous
import functools

import jax
import jax.numpy as jnp
from jax import lax
from jax.experimental import pallas as pl
from jax.experimental.pallas import tpu as pltpu
from jax.experimental.pallas import tpu_sc as plsc

N = 10000
NP = 10240
D = 128
DH = D // 2
K = 10
NTILE = 16
RPT = NP // NTILE
CHUNK = 128
GSUB = 2
SRING = 4
PKBITS = 14
PKMASK = (1 << PKBITS) - 1


def _bn_body(x_ref, g_ref, b_ref, lo_ref, hi_ref):
    x = x_ref[...]
    mu = jnp.mean(x, axis=0, keepdims=True)
    xc = x - mu
    var = jnp.mean(xc * xc, axis=0, keepdims=True)
    xb = xc * (g_ref[...] / jnp.sqrt(var + 1e-5)) + b_ref[...]
    zpad = jnp.zeros((NP - N, D), jnp.float32)
    xbp = jnp.concatenate([xb, zpad], axis=0)
    lo_ref[...] = xbp[:, :DH]
    hi_ref[...] = xbp[:, DH:]


def _batchnorm(x, gamma, beta):
    return pl.pallas_call(
        _bn_body,
        out_shape=(
            jax.ShapeDtypeStruct((NP, DH), jnp.float32),
            jax.ShapeDtypeStruct((NP, DH), jnp.float32),
        ),
    )(x, gamma.reshape(1, D), beta.reshape(1, D))


def _mlp_body(lo_ref, hi_ref, w1a_ref, w1b_ref, b1_ref, a1_ref,
              w2_ref, b2_ref, a2_ref, w3_ref, b3_ref, out_ref):
    f32 = jnp.float32
    h1 = (lax.dot_general(lo_ref[...], w1a_ref[...], (((1,), (0,)), ((), ())),
                          preferred_element_type=f32)
          + lax.dot_general(hi_ref[...], w1b_ref[...], (((1,), (0,)), ((), ())),
                            preferred_element_type=f32)
          + b1_ref[...])
    a1 = a1_ref[0, 0]
    h1 = jnp.where(h1 >= 0, h1, a1 * h1)
    h2 = lax.dot_general(h1, w2_ref[...], (((1,), (0,)), ((), ())),
                         preferred_element_type=f32) + b2_ref[...]
    a2 = a2_ref[0, 0]
    h2 = jnp.where(h2 >= 0, h2, a2 * h2)
    logits = lax.dot_general(h2, w3_ref[...], (((1,), (0,)), ((), ())),
                             preferred_element_type=f32) + b3_ref[...]
    lane = lax.broadcasted_iota(jnp.int32, logits.shape, 1)
    neg = jnp.float32(-1e30)
    masked = jnp.where(lane < 2, logits, neg)
    m = jnp.max(masked, axis=1, keepdims=True)
    e = jnp.where(lane < 2, jnp.exp(masked - m), 0.0)
    lse = m + jnp.log(jnp.sum(e, axis=1, keepdims=True))
    res = logits - lse
    out_ref[...] = res[:, :2]


def _mlp(lo, hi, w1, b1, a1, w2, b2, a2, w3, b3):
    w3p = jnp.zeros((64, 128), jnp.float32).at[:, :2].set(w3)
    b3p = jnp.zeros((1, 128), jnp.float32).at[0, :2].set(b3)
    return pl.pallas_call(
        _mlp_body,
        out_shape=jax.ShapeDtypeStruct((N, 2), jnp.float32),
    )(lo, hi, w1[:DH], w1[DH:], b1.reshape(1, 64), a1.reshape(1, 1),
      w2, b2.reshape(1, 64), a2.reshape(1, 1), w3p, b3p)


def _scale2(g, off, sb, w_v, ch):
    @pl.loop(0, CHUNK // 16)
    def _grp(g16):
        wv16 = w_v[ch, pl.ds(g16 * 16, 16)]
        for e in range(16):
            w1 = jnp.full((16,), wv16[e], jnp.float32)
            j = g16 * 16 + e
            for q in range(DH // 16):
                sl = pl.ds(q * 16, 16)
                sb[j, sl] = g[off + j, sl] * w1


def _unpack_cols(pk_v, c, gidx, slot):
    for q in range(CHUNK // 16):
        sl = pl.ds(q * 16, 16)
        gidx[slot, sl] = jnp.bitwise_and(pk_v[c, sl], PKMASK)


def _unpack_rows(pk_v, c, sidx, slot):
    for q in range(CHUNK // 16):
        sl = pl.ds(q * 16, 16)
        sidx[slot, sl] = jax.lax.shift_right_logical(pk_v[c, sl], PKBITS)


def _prop_tile(x_in, work, zeros_hbm, pk_v, w_v, gidx, sidx,
               gb, sbuf, y_sp, gsem, ssem, s):
    nch = w_v.shape[0]
    rs = pl.ds(s * RPT, RPT)

    for k in range(RPT // CHUNK):
        part = pl.ds(s * RPT + k * CHUNK, CHUNK)
        stage = gb[0].at[pl.ds(0, CHUNK)]
        pltpu.sync_copy(x_in.at[part], stage)
        pltpu.sync_copy(stage, work.at[part])
    plsc.subcore_barrier()

    @pl.loop(0, K)
    def _iter(_):
        pltpu.sync_copy(zeros_hbm.at[rs], y_sp.at[rs])
        plsc.subcore_barrier()

        for b in range(2):
            for u in range(GSUB):
                gslot = b * GSUB + u
                _unpack_cols(pk_v, gslot, gidx, gslot)
                pltpu.async_copy(work.at[gidx.at[gslot]],
                                 gb[b].at[pl.ds(u * CHUNK, CHUNK)], gsem[b])

        @pl.loop(0, nch, step=2 * GSUB)
        def _slots(ch):
            for b in range(2):
                base = ch + b * GSUB
                g = gb[b]
                for u in range(GSUB):
                    pltpu.make_async_copy(
                        work.at[gidx.at[b * GSUB + u]],
                        g.at[pl.ds(u * CHUNK, CHUNK)], gsem[b]).wait()

                for u in range(GSUB):
                    c = base + u
                    p = b * GSUB + u
                    sb = sbuf[p]

                    @pl.when(c >= SRING)
                    def _():
                        pltpu.make_async_copy(
                            sb, y_sp.at[sidx.at[p]], ssem[p]).wait()

                    _scale2(g, u * CHUNK, sb, w_v, c)
                    _unpack_rows(pk_v, c, sidx, p)
                    pltpu.async_copy(sb, y_sp.at[sidx.at[p]], ssem[p],
                                     add=True)

                for u in range(GSUB):
                    gslot = b * GSUB + u
                    _unpack_cols(pk_v, base + 2 * GSUB + u, gidx, gslot)
                    pltpu.async_copy(work.at[gidx.at[gslot]],
                                     g.at[pl.ds(u * CHUNK, CHUNK)], gsem[b])

        for b in range(2):
            for u in range(GSUB):
                pltpu.make_async_copy(
                    work.at[gidx.at[b * GSUB + u]],
                    gb[b].at[pl.ds(u * CHUNK, CHUNK)], gsem[b]).wait()
        for p in range(SRING):
            pltpu.make_async_copy(
                sbuf[p], y_sp.at[sidx.at[p]], ssem[p]).wait()

        plsc.subcore_barrier()
        pltpu.sync_copy(y_sp.at[rs], work.at[rs])
        plsc.subcore_barrier()


def _propagate(xb_lo, xb_hi, pk3, w3, zeros):
    nch = w3.shape[1]
    mesh = plsc.VectorSubcoreMesh(core_axis_name="c", subcore_axis_name="s")

    @functools.partial(
        pl.kernel,
        out_type=(
            jax.ShapeDtypeStruct((NP, DH), jnp.float32),
            jax.ShapeDtypeStruct((NP, DH), jnp.float32),
        ),
        mesh=mesh,
        compiler_params=pltpu.CompilerParams(use_tc_tiling_on_sc=False),
        scratch_types=[
            pltpu.VMEM((nch + 2 * GSUB, CHUNK), jnp.int32),
            pltpu.VMEM((nch, CHUNK), jnp.float32),
            pltpu.VMEM((2 * GSUB, CHUNK), jnp.int32),
            pltpu.VMEM((SRING, CHUNK), jnp.int32),
            [pltpu.VMEM((GSUB * CHUNK, DH), jnp.float32)] * 2,
            [pltpu.VMEM((CHUNK, DH), jnp.float32)] * SRING,
            pltpu.VMEM_SHARED((NP, DH), jnp.float32),
            [pltpu.SemaphoreType.DMA] * 2,
            [pltpu.SemaphoreType.DMA] * SRING,
        ],
    )
    def prop(xlo, xhi, pk_hbm, w_hbm, zeros_hbm,
             out_lo, out_hi, pk_v, w_v, gidx, sidx, gb, sbuf, y_sp,
             gsem, ssem):
        c = lax.axis_index("c")
        s = lax.axis_index("s")
        pltpu.sync_copy(pk_hbm.at[s], pk_v)
        pltpu.sync_copy(w_hbm.at[s], w_v)

        @pl.when(c == 0)
        def _():
            _prop_tile(xlo, out_lo, zeros_hbm, pk_v, w_v, gidx, sidx,
                       gb, sbuf, y_sp, gsem, ssem, s)

        @pl.when(c == 1)
        def _():
            _prop_tile(xhi, out_hi, zeros_hbm, pk_v, w_v, gidx, sidx,
                       gb, sbuf, y_sp, gsem, ssem, s)

    return prop(xb_lo, xb_hi, pk3, w3, zeros)


def kernel(x, x_cov, edge_index, edge_weight, norm_weight, bn_gamma, bn_beta,
           p0_W, p0_C, p1_W, p1_C, feature_corr,
           mlp_W1, mlp_b1, mlp_a1, mlp_W2, mlp_b2, mlp_a2, mlp_W3, mlp_b3):
    E = edge_index.shape[1]
    quant = 2 * GSUB * CHUNK
    ept = ((E + NTILE * quant - 1) // (NTILE * quant)) * quant
    epad = NTILE * ept
    pad = epad - E
    spread = (jnp.arange(pad, dtype=jnp.int32) * 37) % N
    rows = jnp.concatenate([edge_index[0].astype(jnp.int32), spread])
    cols = jnp.concatenate([edge_index[1].astype(jnp.int32), spread])
    w = jnp.concatenate([norm_weight, jnp.zeros((pad,), norm_weight.dtype)])
    nch = ept // CHUNK
    packed = (rows << PKBITS) | cols
    pk3 = packed.reshape(NTILE, nch, CHUNK)
    gpad = (jnp.arange(NTILE * 2 * GSUB * CHUNK, dtype=jnp.int32) * 53) % N
    gpad = gpad.reshape(NTILE, 2 * GSUB, CHUNK)
    pk3 = jnp.concatenate([pk3, (gpad << PKBITS) | gpad], axis=1)
    w3 = w.reshape(NTILE, nch, CHUNK)
    zeros = jnp.zeros((NP, DH), jnp.float32)

    xb_lo, xb_hi = _batchnorm(x, bn_gamma, bn_beta)
    xh_lo, xh_hi = _propagate(xb_lo, xb_hi, pk3, w3, zeros)
    return _mlp(xh_lo[:N], xh_hi[:N], mlp_W1, mlp_b1, mlp_a1,
                mlp_W2, mlp_b2, mlp_a2, mlp_W3, mlp_b3)


# speedup vs baseline: 5.1520x; 1.0253x over previous
import functools

import jax
import jax.numpy as jnp
from jax import lax
from jax.experimental import pallas as pl
from jax.experimental.pallas import tpu as pltpu
from jax.experimental.pallas import tpu_sc as plsc

N = 10000
NP = 10240
D = 128
DH = D // 2
K = 10
NTILE = 16
RPT = NP // NTILE
CHUNK = 128
GSUB = 2
SRING = 4
PKBITS = 14
PKMASK = (1 << PKBITS) - 1


def _bn_body(x_ref, g_ref, b_ref, lo_ref, hi_ref):
    x = x_ref[...]
    mu = jnp.mean(x, axis=0, keepdims=True)
    xc = x - mu
    var = jnp.mean(xc * xc, axis=0, keepdims=True)
    xb = xc * (g_ref[...] / jnp.sqrt(var + 1e-5)) + b_ref[...]
    zpad = jnp.zeros((NP - N, D), jnp.float32)
    xbp = jnp.concatenate([xb, zpad], axis=0)
    lo_ref[...] = xbp[:, :DH]
    hi_ref[...] = xbp[:, DH:]


def _batchnorm(x, gamma, beta):
    return pl.pallas_call(
        _bn_body,
        out_shape=(
            jax.ShapeDtypeStruct((NP, DH), jnp.float32),
            jax.ShapeDtypeStruct((NP, DH), jnp.float32),
        ),
    )(x, gamma.reshape(1, D), beta.reshape(1, D))


def _mlp_body(lo_ref, hi_ref, w1a_ref, w1b_ref, b1_ref, a1_ref,
              w2_ref, b2_ref, a2_ref, w3_ref, b3_ref, out_ref):
    f32 = jnp.float32
    h1 = (lax.dot_general(lo_ref[...], w1a_ref[...], (((1,), (0,)), ((), ())),
                          preferred_element_type=f32)
          + lax.dot_general(hi_ref[...], w1b_ref[...], (((1,), (0,)), ((), ())),
                            preferred_element_type=f32)
          + b1_ref[...])
    a1 = a1_ref[0, 0]
    h1 = jnp.where(h1 >= 0, h1, a1 * h1)
    h2 = lax.dot_general(h1, w2_ref[...], (((1,), (0,)), ((), ())),
                         preferred_element_type=f32) + b2_ref[...]
    a2 = a2_ref[0, 0]
    h2 = jnp.where(h2 >= 0, h2, a2 * h2)
    logits = lax.dot_general(h2, w3_ref[...], (((1,), (0,)), ((), ())),
                             preferred_element_type=f32) + b3_ref[...]
    lane = lax.broadcasted_iota(jnp.int32, logits.shape, 1)
    neg = jnp.float32(-1e30)
    masked = jnp.where(lane < 2, logits, neg)
    m = jnp.max(masked, axis=1, keepdims=True)
    e = jnp.where(lane < 2, jnp.exp(masked - m), 0.0)
    lse = m + jnp.log(jnp.sum(e, axis=1, keepdims=True))
    res = logits - lse
    out_ref[...] = res[:, :2]


def _mlp(lo, hi, w1, b1, a1, w2, b2, a2, w3, b3):
    w3p = jnp.zeros((64, 128), jnp.float32).at[:, :2].set(w3)
    b3p = jnp.zeros((1, 128), jnp.float32).at[0, :2].set(b3)
    return pl.pallas_call(
        _mlp_body,
        out_shape=jax.ShapeDtypeStruct((N, 2), jnp.float32),
    )(lo, hi, w1[:DH], w1[DH:], b1.reshape(1, 64), a1.reshape(1, 1),
      w2, b2.reshape(1, 64), a2.reshape(1, 1), w3p, b3p)


def _scale2(g, off, sb, w_v, ch):
    @pl.loop(0, CHUNK // 16)
    def _grp(g16):
        wv16 = w_v[ch, pl.ds(g16 * 16, 16)]
        for e in range(16):
            w1 = jnp.full((16,), wv16[e], jnp.float32)
            j = g16 * 16 + e
            for q in range(DH // 16):
                sl = pl.ds(q * 16, 16)
                sb[j, sl] = g[off + j, sl] * w1


def _unpack_cols(pk_v, c, gidx, slot):
    for q in range(CHUNK // 16):
        sl = pl.ds(q * 16, 16)
        gidx[slot, sl] = jnp.bitwise_and(pk_v[c, sl], PKMASK)


def _unpack_rows(pk_v, c, sidx, slot):
    for q in range(CHUNK // 16):
        sl = pl.ds(q * 16, 16)
        sidx[slot, sl] = jax.lax.shift_right_logical(pk_v[c, sl], PKBITS)


def _prop_tile(x_in, work, zeros_hbm, pk_v, w_v, gidx, sidx,
               gb, sbuf, y_sp, gsem, ssem, s):
    nch = w_v.shape[0]
    rs = pl.ds(s * RPT, RPT)

    for k in range(RPT // CHUNK):
        part = pl.ds(s * RPT + k * CHUNK, CHUNK)
        stage = gb[0].at[pl.ds(0, CHUNK)]
        pltpu.sync_copy(x_in.at[part], stage)
        pltpu.sync_copy(stage, work.at[part])
    plsc.subcore_barrier()

    @pl.loop(0, K)
    def _iter(_):
        for b in range(2):
            for u in range(GSUB):
                gslot = b * GSUB + u
                _unpack_cols(pk_v, gslot, gidx, gslot)
                pltpu.async_copy(work.at[gidx.at[gslot]],
                                 gb[b].at[pl.ds(u * CHUNK, CHUNK)], gsem[b])

        pltpu.sync_copy(zeros_hbm.at[rs], y_sp.at[rs])
        plsc.subcore_barrier()

        @pl.loop(0, nch, step=2 * GSUB)
        def _slots(ch):
            for b in range(2):
                base = ch + b * GSUB
                g = gb[b]
                for u in range(GSUB):
                    pltpu.make_async_copy(
                        work.at[gidx.at[b * GSUB + u]],
                        g.at[pl.ds(u * CHUNK, CHUNK)], gsem[b]).wait()

                for u in range(GSUB):
                    c = base + u
                    p = b * GSUB + u
                    sb = sbuf[p]

                    @pl.when(c >= SRING)
                    def _():
                        pltpu.make_async_copy(
                            sb, y_sp.at[sidx.at[p]], ssem[p]).wait()

                    _scale2(g, u * CHUNK, sb, w_v, c)
                    _unpack_rows(pk_v, c, sidx, p)
                    pltpu.async_copy(sb, y_sp.at[sidx.at[p]], ssem[p],
                                     add=True)

                for u in range(GSUB):
                    gslot = b * GSUB + u
                    _unpack_cols(pk_v, base + 2 * GSUB + u, gidx, gslot)
                    pltpu.async_copy(work.at[gidx.at[gslot]],
                                     g.at[pl.ds(u * CHUNK, CHUNK)], gsem[b])

        for b in range(2):
            for u in range(GSUB):
                pltpu.make_async_copy(
                    work.at[gidx.at[b * GSUB + u]],
                    gb[b].at[pl.ds(u * CHUNK, CHUNK)], gsem[b]).wait()
        for p in range(SRING):
            pltpu.make_async_copy(
                sbuf[p], y_sp.at[sidx.at[p]], ssem[p]).wait()

        plsc.subcore_barrier()
        pltpu.sync_copy(y_sp.at[rs], work.at[rs])
        plsc.subcore_barrier()


def _propagate(xb_lo, xb_hi, pk3, w3, zeros):
    nch = w3.shape[1]
    mesh = plsc.VectorSubcoreMesh(core_axis_name="c", subcore_axis_name="s")

    @functools.partial(
        pl.kernel,
        out_type=(
            jax.ShapeDtypeStruct((NP, DH), jnp.float32),
            jax.ShapeDtypeStruct((NP, DH), jnp.float32),
        ),
        mesh=mesh,
        compiler_params=pltpu.CompilerParams(use_tc_tiling_on_sc=False),
        scratch_types=[
            pltpu.VMEM((nch + 2 * GSUB, CHUNK), jnp.int32),
            pltpu.VMEM((nch, CHUNK), jnp.float32),
            pltpu.VMEM((2 * GSUB, CHUNK), jnp.int32),
            pltpu.VMEM((SRING, CHUNK), jnp.int32),
            [pltpu.VMEM((GSUB * CHUNK, DH), jnp.float32)] * 2,
            [pltpu.VMEM((CHUNK, DH), jnp.float32)] * SRING,
            pltpu.VMEM_SHARED((NP, DH), jnp.float32),
            [pltpu.SemaphoreType.DMA] * 2,
            [pltpu.SemaphoreType.DMA] * SRING,
        ],
    )
    def prop(xlo, xhi, pk_hbm, w_hbm, zeros_hbm,
             out_lo, out_hi, pk_v, w_v, gidx, sidx, gb, sbuf, y_sp,
             gsem, ssem):
        c = lax.axis_index("c")
        s = lax.axis_index("s")
        pltpu.sync_copy(pk_hbm.at[s], pk_v)
        pltpu.sync_copy(w_hbm.at[s], w_v)

        @pl.when(c == 0)
        def _():
            _prop_tile(xlo, out_lo, zeros_hbm, pk_v, w_v, gidx, sidx,
                       gb, sbuf, y_sp, gsem, ssem, s)

        @pl.when(c == 1)
        def _():
            _prop_tile(xhi, out_hi, zeros_hbm, pk_v, w_v, gidx, sidx,
                       gb, sbuf, y_sp, gsem, ssem, s)

    return prop(xb_lo, xb_hi, pk3, w3, zeros)


def kernel(x, x_cov, edge_index, edge_weight, norm_weight, bn_gamma, bn_beta,
           p0_W, p0_C, p1_W, p1_C, feature_corr,
           mlp_W1, mlp_b1, mlp_a1, mlp_W2, mlp_b2, mlp_a2, mlp_W3, mlp_b3):
    E = edge_index.shape[1]
    quant = 2 * GSUB * CHUNK
    ept = ((E + NTILE * quant - 1) // (NTILE * quant)) * quant
    epad = NTILE * ept
    pad = epad - E
    spread = (jnp.arange(pad, dtype=jnp.int32) * 37) % N
    rows = jnp.concatenate([edge_index[0].astype(jnp.int32), spread])
    cols = jnp.concatenate([edge_index[1].astype(jnp.int32), spread])
    w = jnp.concatenate([norm_weight, jnp.zeros((pad,), norm_weight.dtype)])
    nch = ept // CHUNK
    packed = (rows << PKBITS) | cols
    pk3 = packed.reshape(NTILE, nch, CHUNK)
    gpad = (jnp.arange(NTILE * 2 * GSUB * CHUNK, dtype=jnp.int32) * 53) % N
    gpad = gpad.reshape(NTILE, 2 * GSUB, CHUNK)
    pk3 = jnp.concatenate([pk3, (gpad << PKBITS) | gpad], axis=1)
    w3 = w.reshape(NTILE, nch, CHUNK)
    zeros = jnp.zeros((NP, DH), jnp.float32)

    xb_lo, xb_hi = _batchnorm(x, bn_gamma, bn_beta)
    xh_lo, xh_hi = _propagate(xb_lo, xb_hi, pk3, w3, zeros)
    return _mlp(xh_lo[:N], xh_hi[:N], mlp_W1, mlp_b1, mlp_a1,
                mlp_W2, mlp_b2, mlp_a2, mlp_W3, mlp_b3)
